# baseline (speedup 1.0000x reference)
import functools

import jax
import jax.numpy as jnp
from jax import lax
from jax.experimental import pallas as pl
from jax.experimental.pallas import tpu as pltpu
from jax.experimental.pallas import tpu_sc as plsc

USER_N = 4000
ITEM_N = 6000
NN = USER_N + ITEM_N
EE = 320000
DD = 128
LANE = 16
DV = DD // LANE

NC = 2
NS = 16
NW = NC * NS

CHUNK = 80
NCHUNKS = EE // CHUNK
NBUF = 4
UNIT = 40
NUNITS = NN // UNIT

RCH = 80
NRCH = NN // RCH

_MESH = plsc.VectorSubcoreMesh(core_axis_name="c", subcore_axis_name="s", num_cores=2, num_subcores=16)


@functools.partial(
    pl.kernel,
    out_type=jax.ShapeDtypeStruct((NC, NN, DD), jnp.float32),
    mesh=_MESH,
    compiler_params=pltpu.CompilerParams(needs_layout_passes=False),
    scratch_types=[
        [pltpu.VMEM((CHUNK,), jnp.int32) for _ in range(NBUF)],
        [pltpu.VMEM((CHUNK,), jnp.int32) for _ in range(NBUF)],
        [pltpu.VMEM((CHUNK,), jnp.float32) for _ in range(NBUF)],
        [pltpu.VMEM((CHUNK, DD), jnp.float32) for _ in range(NBUF)],
        [pltpu.VMEM((CHUNK,), jnp.int32) for _ in range(NBUF)],
        pltpu.VMEM_SHARED((NN, DD), jnp.float32),
        [pltpu.SemaphoreType.DMA for _ in range(NBUF)],
        [pltpu.SemaphoreType.DMA for _ in range(NBUF)],
        [pltpu.SemaphoreType.DMA for _ in range(NBUF)],
        pltpu.SemaphoreType.DMA,
    ],
)
def _spmm(row_hbm, col_hbm, val_hbm, x_hbm, z_hbm, out_hbm, rbuf, cbuf, vbuf,
          xbuf, rowc, acc, isem, gsem, ssem, wsem):
    cid = lax.axis_index("c")
    sid = lax.axis_index("s")
    wid = sid * NC + cid
    n = (NCHUNKS - 1 - wid) // NW + 1

    def chunk_off(i):
        return (wid + i * NW) * CHUNK

    def idx_start(i, b):
        off = chunk_off(i)
        pltpu.make_async_copy(row_hbm.at[pl.ds(off, CHUNK)], rbuf[b],
                              isem[b]).start()
        pltpu.make_async_copy(col_hbm.at[pl.ds(off, CHUNK)], cbuf[b],
                              isem[b]).start()
        pltpu.make_async_copy(val_hbm.at[pl.ds(off, CHUNK)], vbuf[b],
                              isem[b]).start()

    def idx_wait(b):
        pltpu.make_async_copy(row_hbm.at[pl.ds(0, CHUNK)], rbuf[b],
                              isem[b]).wait()
        pltpu.make_async_copy(col_hbm.at[pl.ds(0, CHUNK)], cbuf[b],
                              isem[b]).wait()
        pltpu.make_async_copy(val_hbm.at[pl.ds(0, CHUNK)], vbuf[b],
                              isem[b]).wait()

    def gather_start(b):
        pltpu.make_async_copy(x_hbm.at[cbuf[b]], xbuf[b],
                              gsem[b]).start()

    def gather_wait(b):
        pltpu.make_async_copy(x_hbm.at[cbuf[b]], xbuf[b],
                              gsem[b]).wait()

    def scatter_start(b):
        pltpu.make_async_copy(xbuf[b], acc.at[rowc[b]],
                              ssem[b]).start(add=True)

    def scatter_wait(b):
        pltpu.make_async_copy(xbuf[b], acc.at[rowc[b]], ssem[b]).wait()

    idx_start(0, 0)
    idx_start(1, 1)
    idx_start(2, 2)
    idx_wait(0)
    gather_start(0)
    idx_wait(1)
    gather_start(1)

    n_my_units = (NUNITS - 1 - sid) // NS + 1

    def zunit(k, carry):
        u = sid + k * NS
        pltpu.make_async_copy(z_hbm, acc.at[pl.ds(u * UNIT, UNIT)],
                              wsem).start()
        return carry

    lax.fori_loop(0, n_my_units, zunit, 0)

    def zdrain(k, carry):
        pltpu.make_async_copy(z_hbm, acc.at[pl.ds(0, UNIT)], wsem).wait()
        return carry

    lax.fori_loop(0, n_my_units, zdrain, 0)
    plsc.subcore_barrier()

    def body(t, carry):
        for b in range(NBUF):
            i = NBUF * t + b
            bnn = (b + 2) % NBUF
            bnnn = (b + 3) % NBUF

            @pl.when(i + 3 < n)
            def _():
                idx_start(i + 3, bnnn)

            @pl.when(i + 2 < n)
            def _():
                @pl.when(i >= 2)
                def _():
                    scatter_wait(bnn)

                idx_wait(bnn)
                gather_start(bnn)

            @pl.when(i < n)
            def _():
                gather_wait(b)
                for g in range(CHUNK // LANE):
                    rowc[b][pl.ds(g * LANE, LANE)] = (
                        rbuf[b][pl.ds(g * LANE, LANE)])

                @plsc.parallel_loop(0, CHUNK, step=LANE)
                def mul(j0):
                    v16 = vbuf[b][pl.ds(j0, LANE)]
                    for jj in range(LANE):
                        vv = jnp.take_along_axis(
                            v16, jnp.full((LANE,), jj, jnp.int32), axis=0,
                            mode="promise_in_bounds")
                        for d in range(DV):
                            xbuf[b][j0 + jj, pl.ds(d * LANE, LANE)] = (
                                xbuf[b][j0 + jj, pl.ds(d * LANE, LANE)] * vv)

                scatter_start(b)

        return carry

    lax.fori_loop(0, (n + NBUF - 1) // NBUF, body, 0)
    for b in range(NBUF):
        scatter_wait(b)

    plsc.subcore_barrier()

    def wunit(k, carry):
        u = sid + k * NS
        pltpu.make_async_copy(acc.at[pl.ds(u * UNIT, UNIT)],
                              out_hbm.at[cid, pl.ds(u * UNIT, UNIT)],
                              wsem).start()
        return carry

    lax.fori_loop(0, n_my_units, wunit, 0)

    def wdrain(k, carry):
        pltpu.make_async_copy(acc.at[pl.ds(0, UNIT)],
                              out_hbm.at[cid, pl.ds(0, UNIT)], wsem).wait()
        return carry

    lax.fori_loop(0, n_my_units, wdrain, 0)


@functools.partial(
    pl.kernel,
    out_type=jax.ShapeDtypeStruct((NN, DD), jnp.float32),
    mesh=_MESH,
    compiler_params=pltpu.CompilerParams(needs_layout_passes=False),
    scratch_types=[
        pltpu.VMEM((RCH, DD), jnp.float32),
        pltpu.VMEM((RCH, DD), jnp.float32),
        pltpu.SemaphoreType.DMA,
    ],
)
def _combine(p_hbm, out_hbm, b0, b1, csem):
    cid = lax.axis_index("c")
    sid = lax.axis_index("s")
    wid = sid * NC + cid
    n_my = (NRCH - 1 - wid) // NW + 1

    def body(i, carry):
        r0 = (wid + i * NW) * RCH
        pltpu.make_async_copy(p_hbm.at[0, pl.ds(r0, RCH)], b0, csem).start()
        pltpu.make_async_copy(p_hbm.at[1, pl.ds(r0, RCH)], b1, csem).start()
        pltpu.make_async_copy(p_hbm.at[0, pl.ds(r0, RCH)], b0, csem).wait()
        pltpu.make_async_copy(p_hbm.at[1, pl.ds(r0, RCH)], b1, csem).wait()

        @plsc.parallel_loop(0, RCH, unroll=8)
        def addrow(j):
            for d in range(DV):
                b0[j, pl.ds(d * LANE, LANE)] = (
                    b0[j, pl.ds(d * LANE, LANE)]
                    + b1[j, pl.ds(d * LANE, LANE)])

        pltpu.sync_copy(b0, out_hbm.at[pl.ds(r0, RCH)])
        return carry

    lax.fori_loop(0, n_my, body, 0)


@functools.partial(
    pl.kernel,
    out_type=jax.ShapeDtypeStruct((NN, DD), jnp.float32),
    mesh=_MESH,
    compiler_params=pltpu.CompilerParams(needs_layout_passes=False),
    scratch_types=[
        [pltpu.VMEM((RCH, DD), jnp.float32) for _ in range(5)],
        pltpu.SemaphoreType.DMA,
    ],
)
def _final(x0_hbm, x1_hbm, x2_hbm, p_hbm, out_hbm, bufs, fsem):
    cid = lax.axis_index("c")
    sid = lax.axis_index("s")
    wid = sid * NC + cid
    n_my = (NRCH - 1 - wid) // NW + 1

    def body(i, carry):
        r0 = (wid + i * NW) * RCH
        srcs = [x0_hbm.at[pl.ds(r0, RCH)], x1_hbm.at[pl.ds(r0, RCH)],
                x2_hbm.at[pl.ds(r0, RCH)], p_hbm.at[0, pl.ds(r0, RCH)],
                p_hbm.at[1, pl.ds(r0, RCH)]]
        for s, bb in zip(srcs, bufs):
            pltpu.make_async_copy(s, bb, fsem).start()
        for s, bb in zip(srcs, bufs):
            pltpu.make_async_copy(s, bb, fsem).wait()

        @plsc.parallel_loop(0, RCH, unroll=8)
        def addrow(j):
            for d in range(DV):
                acc = bufs[0][j, pl.ds(d * LANE, LANE)]
                for bb in bufs[1:]:
                    acc = acc + bb[j, pl.ds(d * LANE, LANE)]
                bufs[0][j, pl.ds(d * LANE, LANE)] = acc * 0.25

        pltpu.sync_copy(bufs[0], out_hbm.at[pl.ds(r0, RCH)])
        return carry

    lax.fori_loop(0, n_my, body, 0)


def kernel(edge_index, edge_vals, uEmbeds, iEmbeds):
    row = edge_index[0]
    col = edge_index[1]
    x0 = jnp.concatenate([uEmbeds, iEmbeds], axis=0)
    zunit = jnp.zeros((UNIT, DD), jnp.float32)
    p1 = _spmm(row, col, edge_vals, x0, zunit)
    x1 = _combine(p1)
    p2 = _spmm(row, col, edge_vals, x1, zunit)
    x2 = _combine(p2)
    p3 = _spmm(row, col, edge_vals, x2, zunit)
    main = _final(x0, x1, x2, p3)
    return main[:USER_N], main[USER_N:]


# speedup vs baseline: 1.0408x; 1.0408x over previous
import functools

import jax
import jax.numpy as jnp
from jax import lax
from jax.experimental import pallas as pl
from jax.experimental.pallas import tpu as pltpu
from jax.experimental.pallas import tpu_sc as plsc

USER_N = 4000
ITEM_N = 6000
NN = USER_N + ITEM_N
EE = 320000
DD = 128
LANE = 16
DV = DD // LANE

NC = 2
NS = 16
NW = NC * NS

CHUNK = 80
NCHUNKS = EE // CHUNK
NBUF = 4
UNIT = 40
NUNITS = NN // UNIT

RCH = 80
NRCH = NN // RCH

_MESH = plsc.VectorSubcoreMesh(core_axis_name="c", subcore_axis_name="s", num_cores=2, num_subcores=16)


@functools.partial(
    pl.kernel,
    out_type=jax.ShapeDtypeStruct((NC, NN, DD), jnp.float32),
    mesh=_MESH,
    compiler_params=pltpu.CompilerParams(needs_layout_passes=False),
    scratch_types=[
        [pltpu.VMEM((CHUNK,), jnp.int32) for _ in range(NBUF)],
        [pltpu.VMEM((CHUNK,), jnp.int32) for _ in range(NBUF)],
        [pltpu.VMEM((CHUNK,), jnp.float32) for _ in range(NBUF)],
        [pltpu.VMEM((CHUNK, DD), jnp.float32) for _ in range(NBUF)],
        [pltpu.VMEM((CHUNK,), jnp.int32) for _ in range(NBUF)],
        pltpu.VMEM_SHARED((NN, DD), jnp.float32),
        [pltpu.SemaphoreType.DMA for _ in range(NBUF)],
        [pltpu.SemaphoreType.DMA for _ in range(NBUF)],
        [pltpu.SemaphoreType.DMA for _ in range(NBUF)],
        pltpu.SemaphoreType.DMA,
    ],
)
def _spmm(row_hbm, col_hbm, val_hbm, x_hbm, z_hbm, out_hbm, rbuf, cbuf, vbuf,
          xbuf, rowc, acc, isem, gsem, ssem, wsem):
    cid = lax.axis_index("c")
    sid = lax.axis_index("s")
    wid = sid * NC + cid
    n = (NCHUNKS - 1 - wid) // NW + 1

    def chunk_off(i):
        return (wid + i * NW) * CHUNK

    def idx_start(i, b):
        off = chunk_off(i)
        pltpu.make_async_copy(row_hbm.at[pl.ds(off, CHUNK)], rbuf[b],
                              isem[b]).start()
        pltpu.make_async_copy(col_hbm.at[pl.ds(off, CHUNK)], cbuf[b],
                              isem[b]).start()
        pltpu.make_async_copy(val_hbm.at[pl.ds(off, CHUNK)], vbuf[b],
                              isem[b]).start()

    def idx_wait(b):
        pltpu.make_async_copy(row_hbm.at[pl.ds(0, CHUNK)], rbuf[b],
                              isem[b]).wait()
        pltpu.make_async_copy(col_hbm.at[pl.ds(0, CHUNK)], cbuf[b],
                              isem[b]).wait()
        pltpu.make_async_copy(val_hbm.at[pl.ds(0, CHUNK)], vbuf[b],
                              isem[b]).wait()

    def gather_start(b):
        pltpu.make_async_copy(x_hbm.at[cbuf[b]], xbuf[b],
                              gsem[b]).start()

    def gather_wait(b):
        pltpu.make_async_copy(x_hbm.at[cbuf[b]], xbuf[b],
                              gsem[b]).wait()

    def scatter_start(b):
        pltpu.make_async_copy(xbuf[b], acc.at[rowc[b]],
                              ssem[b]).start(add=True)

    def scatter_wait(b):
        pltpu.make_async_copy(xbuf[b], acc.at[rowc[b]], ssem[b]).wait()

    idx_start(0, 0)
    idx_start(1, 1)
    idx_start(2, 2)
    idx_wait(0)
    gather_start(0)
    idx_wait(1)
    gather_start(1)

    n_my_units = (NUNITS - 1 - sid) // NS + 1

    def zunit(k, carry):
        u = sid + k * NS
        pltpu.make_async_copy(z_hbm, acc.at[pl.ds(u * UNIT, UNIT)],
                              wsem).start()
        return carry

    lax.fori_loop(0, n_my_units, zunit, 0)

    def zdrain(k, carry):
        pltpu.make_async_copy(z_hbm, acc.at[pl.ds(0, UNIT)], wsem).wait()
        return carry

    lax.fori_loop(0, n_my_units, zdrain, 0)
    plsc.subcore_barrier()

    def body(t, carry):
        for b in range(NBUF):
            i = NBUF * t + b
            bnn = (b + 2) % NBUF
            bnnn = (b + 3) % NBUF

            @pl.when(i + 3 < n)
            def _():
                idx_start(i + 3, bnnn)

            @pl.when(i + 2 < n)
            def _():
                @pl.when(i >= 2)
                def _():
                    scatter_wait(bnn)

                idx_wait(bnn)
                gather_start(bnn)

            @pl.when(i < n)
            def _():
                gather_wait(b)
                for g in range(CHUNK // LANE):
                    rowc[b][pl.ds(g * LANE, LANE)] = (
                        rbuf[b][pl.ds(g * LANE, LANE)])

                @plsc.parallel_loop(0, CHUNK, unroll=8)
                def mul(j):
                    vv = plsc.load_gather(
                        vbuf[b], [jnp.full((LANE,), j, jnp.int32)])
                    for d in range(DV):
                        xbuf[b][j, pl.ds(d * LANE, LANE)] = (
                            xbuf[b][j, pl.ds(d * LANE, LANE)] * vv)

                scatter_start(b)

        return carry

    lax.fori_loop(0, (n + NBUF - 1) // NBUF, body, 0)
    for b in range(NBUF):
        scatter_wait(b)

    plsc.subcore_barrier()

    def wunit(k, carry):
        u = sid + k * NS
        pltpu.make_async_copy(acc.at[pl.ds(u * UNIT, UNIT)],
                              out_hbm.at[cid, pl.ds(u * UNIT, UNIT)],
                              wsem).start()
        return carry

    lax.fori_loop(0, n_my_units, wunit, 0)

    def wdrain(k, carry):
        pltpu.make_async_copy(acc.at[pl.ds(0, UNIT)],
                              out_hbm.at[cid, pl.ds(0, UNIT)], wsem).wait()
        return carry

    lax.fori_loop(0, n_my_units, wdrain, 0)


@functools.partial(
    pl.kernel,
    out_type=jax.ShapeDtypeStruct((NN, DD), jnp.float32),
    mesh=_MESH,
    compiler_params=pltpu.CompilerParams(needs_layout_passes=False),
    scratch_types=[
        pltpu.VMEM((RCH, DD), jnp.float32),
        pltpu.VMEM((RCH, DD), jnp.float32),
        pltpu.SemaphoreType.DMA,
    ],
)
def _combine(p_hbm, out_hbm, b0, b1, csem):
    cid = lax.axis_index("c")
    sid = lax.axis_index("s")
    wid = sid * NC + cid
    n_my = (NRCH - 1 - wid) // NW + 1

    def body(i, carry):
        r0 = (wid + i * NW) * RCH
        pltpu.make_async_copy(p_hbm.at[0, pl.ds(r0, RCH)], b0, csem).start()
        pltpu.make_async_copy(p_hbm.at[1, pl.ds(r0, RCH)], b1, csem).start()
        pltpu.make_async_copy(p_hbm.at[0, pl.ds(r0, RCH)], b0, csem).wait()
        pltpu.make_async_copy(p_hbm.at[1, pl.ds(r0, RCH)], b1, csem).wait()

        @plsc.parallel_loop(0, RCH, unroll=8)
        def addrow(j):
            for d in range(DV):
                b0[j, pl.ds(d * LANE, LANE)] = (
                    b0[j, pl.ds(d * LANE, LANE)]
                    + b1[j, pl.ds(d * LANE, LANE)])

        pltpu.sync_copy(b0, out_hbm.at[pl.ds(r0, RCH)])
        return carry

    lax.fori_loop(0, n_my, body, 0)


@functools.partial(
    pl.kernel,
    out_type=jax.ShapeDtypeStruct((NN, DD), jnp.float32),
    mesh=_MESH,
    compiler_params=pltpu.CompilerParams(needs_layout_passes=False),
    scratch_types=[
        [pltpu.VMEM((RCH, DD), jnp.float32) for _ in range(5)],
        pltpu.SemaphoreType.DMA,
    ],
)
def _final(x0_hbm, x1_hbm, x2_hbm, p_hbm, out_hbm, bufs, fsem):
    cid = lax.axis_index("c")
    sid = lax.axis_index("s")
    wid = sid * NC + cid
    n_my = (NRCH - 1 - wid) // NW + 1

    def body(i, carry):
        r0 = (wid + i * NW) * RCH
        srcs = [x0_hbm.at[pl.ds(r0, RCH)], x1_hbm.at[pl.ds(r0, RCH)],
                x2_hbm.at[pl.ds(r0, RCH)], p_hbm.at[0, pl.ds(r0, RCH)],
                p_hbm.at[1, pl.ds(r0, RCH)]]
        for s, bb in zip(srcs, bufs):
            pltpu.make_async_copy(s, bb, fsem).start()
        for s, bb in zip(srcs, bufs):
            pltpu.make_async_copy(s, bb, fsem).wait()

        @plsc.parallel_loop(0, RCH, unroll=8)
        def addrow(j):
            for d in range(DV):
                acc = bufs[0][j, pl.ds(d * LANE, LANE)]
                for bb in bufs[1:]:
                    acc = acc + bb[j, pl.ds(d * LANE, LANE)]
                bufs[0][j, pl.ds(d * LANE, LANE)] = acc * 0.25

        pltpu.sync_copy(bufs[0], out_hbm.at[pl.ds(r0, RCH)])
        return carry

    lax.fori_loop(0, n_my, body, 0)


def kernel(edge_index, edge_vals, uEmbeds, iEmbeds):
    row = edge_index[0]
    col = edge_index[1]
    x0 = jnp.concatenate([uEmbeds, iEmbeds], axis=0)
    zunit = jnp.zeros((UNIT, DD), jnp.float32)
    p1 = _spmm(row, col, edge_vals, x0, zunit)
    x1 = _combine(p1)
    p2 = _spmm(row, col, edge_vals, x1, zunit)
    x2 = _combine(p2)
    p3 = _spmm(row, col, edge_vals, x2, zunit)
    main = _final(x0, x1, x2, p3)
    return main[:USER_N], main[USER_N:]
